# baseline (speedup 1.0000x reference)
import functools
import math

import jax
import jax.numpy as jnp
from jax import lax
from jax.experimental import pallas as pl

_N = 256
_SB = 64
_G = 4
_CUTOFF = 10.0
_COEF = math.sqrt(2.0 / _CUTOFF)
_EPS = 1e-8


def _dot(a, b):
    return jnp.dot(a, b, precision=lax.Precision.HIGHEST)


def _tc_body(nuc_ref, ch_ref, f_ref, kf_ref, emb_ref,
             bd0_ref, bd1_ref,
             m0s_ref, m0r_ref, m0e_ref, m0b0_ref, m0w1_ref, m0b1_ref,
             u0n_ref, u0m_ref, u0b0_ref, u0w1_ref, u0b1_ref,
             m1s_ref, m1r_ref, m1e_ref, m1b0_ref, m1w1_ref, m1b1_ref,
             u1n_ref, u1m_ref, u1b0_ref, u1w1_ref, u1b1_ref,
             n0w0_ref, n0b0_ref, n0w1_ref, ne0_ref,
             n1w0_ref, n1b0_ref, n1w1_ref, ne1_ref,
             gw0_ref, gb0_ref, gw1_ref, gb1_ref,
             o0_ref, o1_ref, og_ref):
    nuc = nuc_ref[...]
    ch = ch_ref[...]
    f = f_ref[...]
    f3 = (f * (1.0 / _CUTOFF))[None]
    kf3 = kf_ref[...][None]

    onehot = (lax.broadcasted_iota(jnp.int32, (_N, 8), 1) == ch)
    onehot = onehot.astype(jnp.float32)
    n0 = _dot(onehot, emb_ref[...])

    def silu(x):
        return x * jax.nn.sigmoid(x)

    arg_blocks = []
    for g in range(_G):
        s0 = g * _SB
        d2 = None
        for ax in range(3):
            xs = nuc[s0:s0 + _SB, ax:ax + 1]
            xr = nuc[:, ax:ax + 1].reshape(1, _N)
            dx = xs - xr
            d2 = dx * dx if d2 is None else d2 + dx * dx
        xe3 = (jnp.sqrt(d2) + _EPS).reshape(_SB, _N, 1)
        arg_blocks.append(xe3 * f3)
    argc = jnp.concatenate(arg_blocks, axis=2)
    ec = (jnp.sin(argc) / argc) * kf3
    ef = ec.reshape(_SB * _N, _G * 32)

    ed = jnp.sin(f * (_EPS / _CUTOFF)) * (_COEF / _EPS)

    def layer(n, ws, wr, we, b0, bd, w1, b1, un, um, ub0, uw1, ub1):
        A = _dot(n, ws) + b0
        B = _dot(n, wr)
        A_cat = jnp.concatenate(
            [A[g * _SB:(g + 1) * _SB] for g in range(_G)], axis=1)
        B_cat = jnp.concatenate([B] * _G, axis=1)
        ep = _dot(ef, bd).reshape(_SB, _N, _G * 32)
        pre = ep + A_cat[:, None, :] + B_cat[None]
        sil = pre * jax.nn.sigmoid(pre)
        mc = jnp.sum(sil, axis=1)
        msum = jnp.concatenate(
            [mc[:, g * 32:(g + 1) * 32] for g in range(_G)], axis=0)
        pre_d = A + B + _dot(ed, we)
        msum = msum - silu(pre_d)
        msg = _dot(msum / 255.0, w1) + b1
        t = _dot(n, un) + _dot(msg, um) + ub0
        return _dot(silu(t), uw1) + ub1

    n1 = layer(n0, m0s_ref[...], m0r_ref[...], m0e_ref[...], m0b0_ref[...],
               bd0_ref[...], m0w1_ref[...], m0b1_ref[...],
               u0n_ref[...], u0m_ref[...], u0b0_ref[...],
               u0w1_ref[...], u0b1_ref[...])
    n2 = n1 + layer(n1, m1s_ref[...], m1r_ref[...], m1e_ref[...],
                    m1b0_ref[...], bd1_ref[...], m1w1_ref[...], m1b1_ref[...],
                    u1n_ref[...], u1m_ref[...], u1b0_ref[...],
                    u1w1_ref[...], u1b1_ref[...])

    h = jnp.concatenate([n0, n1, n2], axis=1)

    o0 = (_dot(silu(_dot(h, n0w0_ref[...]) + n0b0_ref[...]),
                  n0w1_ref[...])
          + _dot(onehot, ne0_ref[...]))
    o1 = (_dot(silu(_dot(h, n1w0_ref[...]) + n1b0_ref[...]),
                  n1w1_ref[...])
          + _dot(onehot, ne1_ref[...]))
    gi = jnp.mean(h, axis=0, keepdims=True)
    og = (_dot(silu(_dot(gi, gw0_ref[...]) + gb0_ref[...]),
                  gw1_ref[...])
          + gb1_ref[...])

    o0_ref[...] = o0
    o1_ref[...] = o1
    og_ref[...] = og


@functools.partial(jax.jit, static_argnames=())
def kernel(nuclei, params, charges):
    p = params
    m0W0 = p['mp0']['W0']
    m1W0 = p['mp1']['W0']
    u0W0 = p['up0']['W0']
    u1W0 = p['up1']['W0']

    def row(b):
        return b.reshape(1, -1)

    f_row = row(p['rbf_f'])
    kf_cat = jnp.tile(f_row * (_COEF / _CUTOFF), (1, _G))
    eye_g = jnp.eye(_G, dtype=jnp.float32)
    bd0 = jnp.kron(eye_g, m0W0[64:96])
    bd1 = jnp.kron(eye_g, m1W0[128:160])

    args = (
        nuclei.reshape(_N, 3),
        jnp.clip(charges.reshape(_N, 1).astype(jnp.int32), 0, 7),
        f_row, kf_cat,
        p['embed'],
        bd0, bd1,
        m0W0[0:32], m0W0[32:64], m0W0[64:96], row(p['mp0']['b0']),
        p['mp0']['W1'], row(p['mp0']['b1']),
        u0W0[0:32], u0W0[32:64], row(p['up0']['b0']),
        p['up0']['W1'], row(p['up0']['b1']),
        m1W0[0:64], m1W0[64:128], m1W0[128:160], row(p['mp1']['b0']),
        p['mp1']['W1'], row(p['mp1']['b1']),
        u1W0[0:64], u1W0[64:96], row(p['up1']['b0']),
        p['up1']['W1'], row(p['up1']['b1']),
        p['node0_mlp']['W0'], row(p['node0_mlp']['b0']),
        p['node0_mlp']['W1'], p['node0_embed'],
        p['node1_mlp']['W0'], row(p['node1_mlp']['b0']),
        p['node1_mlp']['W1'], p['node1_embed'],
        p['glob0_mlp']['W0'], row(p['glob0_mlp']['b0']),
        p['glob0_mlp']['W1'], row(p['glob0_mlp']['b1']),
    )

    o0, o1, og = pl.pallas_call(
        _tc_body,
        out_shape=(
            jax.ShapeDtypeStruct((_N, 1), jnp.float32),
            jax.ShapeDtypeStruct((_N, 3), jnp.float32),
            jax.ShapeDtypeStruct((1, 1), jnp.float32),
        ),
    )(*args)
    return (o0, o1, og.reshape(1))


# speedup vs baseline: 1.2598x; 1.2598x over previous
import functools
import math

import jax
import jax.numpy as jnp
from jax import lax
from jax.experimental import pallas as pl

_N = 256
_SB = 64
_G = 4
_CUTOFF = 10.0
_COEF = math.sqrt(2.0 / _CUTOFF)
_EPS = 1e-8


def _bf(x):
    return x.astype(jnp.bfloat16).astype(jnp.float32)


def _dot(a, b):
    return jnp.dot(_bf(a), _bf(b))


def _dot_hi(a, b):
    return jnp.dot(a, b, precision=lax.Precision.HIGHEST)


def _tc_body(nuc_ref, ch_ref, f_ref, fcat_ref, coef_ref, emb_ref,
             bd0_ref, bd1_ref, bdw0_ref, bdw1_ref, b1c0_ref, b1c1_ref,
             m0s_ref, m0r_ref, m0e_ref, m0b0_ref, m0w1_ref, m0b1_ref,
             u0n_ref, u0m_ref, u0b0_ref, u0w1_ref, u0b1_ref,
             m1s_ref, m1r_ref, m1e_ref, m1b0_ref, m1w1_ref, m1b1_ref,
             u1n_ref, u1m_ref, u1b0_ref, u1w1_ref, u1b1_ref,
             n0w0_ref, n0b0_ref, n0w1_ref, ne0_ref,
             n1w0_ref, n1b0_ref, n1w1_ref, ne1_ref,
             gw0_ref, gb0_ref, gw1_ref, gb1_ref,
             o0_ref, o1_ref, og_ref):
    nuc = nuc_ref[...]
    ch = ch_ref[...]
    f = f_ref[...]
    fcat3 = fcat_ref[...][None]
    coef = coef_ref[0, 0]

    onehot = (lax.broadcasted_iota(jnp.int32, (_N, 8), 1) == ch)
    onehot = onehot.astype(jnp.float32)
    n0 = _dot_hi(onehot, emb_ref[...])

    def silu(x):
        return x * jax.nn.sigmoid(x)

    ones32 = jnp.ones((1, 1, 32), jnp.float32)
    xe_blocks = []
    for g in range(_G):
        s0 = g * _SB
        d2 = None
        for ax in range(3):
            xs = nuc[s0:s0 + _SB, ax:ax + 1]
            xr = nuc[:, ax:ax + 1].reshape(1, _N)
            dx = xs - xr
            d2 = dx * dx if d2 is None else d2 + dx * dx
        xe3 = (jnp.sqrt(d2) + _EPS).reshape(_SB, _N, 1)
        xe_blocks.append(xe3 * ones32)
    xec = jnp.concatenate(xe_blocks, axis=2)
    argc = (xec * fcat3) / _CUTOFF
    ec = (coef * jnp.sin(argc)) / xec
    ef = ec.reshape(_SB * _N, _G * 32)

    ed = (coef * jnp.sin((f * _EPS) / _CUTOFF)) / _EPS

    def layer(n, ws, wr, we, b0, bd, bdw1, b1cat, w1, b1,
              un, um, ub0, uw1, ub1):
        A = _dot(n, ws) + b0
        B = _dot(n, wr)
        A_cat = jnp.concatenate(
            [A[g * _SB:(g + 1) * _SB] for g in range(_G)], axis=1)
        B_cat = jnp.concatenate([B] * _G, axis=1)
        ep = _dot(ef, bd).reshape(_SB, _N, _G * 32)
        pre = ep + A_cat[:, None, :] + B_cat[None]
        sil = pre * jax.nn.sigmoid(pre)
        m_cat = _dot(sil.reshape(_SB * _N, _G * 32), bdw1) + b1cat
        mc = jnp.sum(m_cat.reshape(_SB, _N, _G * 32), axis=1)
        msum = jnp.concatenate(
            [mc[:, g * 32:(g + 1) * 32] for g in range(_G)], axis=0)
        pre_d = _dot(ed, we) + A + B
        m_diag = _dot(silu(pre_d), w1) + b1
        msg = (msum - m_diag) / 255.0
        t = _dot(n, un) + _dot(msg, um) + ub0
        return _dot(silu(t), uw1) + ub1

    n1 = layer(n0, m0s_ref[...], m0r_ref[...], m0e_ref[...], m0b0_ref[...],
               bd0_ref[...], bdw0_ref[...], b1c0_ref[...],
               m0w1_ref[...], m0b1_ref[...],
               u0n_ref[...], u0m_ref[...], u0b0_ref[...],
               u0w1_ref[...], u0b1_ref[...])
    n2 = n1 + layer(n1, m1s_ref[...], m1r_ref[...], m1e_ref[...],
                    m1b0_ref[...], bd1_ref[...], bdw1_ref[...], b1c1_ref[...],
                    m1w1_ref[...], m1b1_ref[...],
                    u1n_ref[...], u1m_ref[...], u1b0_ref[...],
                    u1w1_ref[...], u1b1_ref[...])

    h = jnp.concatenate([n0, n1, n2], axis=1)

    o0 = (_dot_hi(silu(_dot(h, n0w0_ref[...]) + n0b0_ref[...]),
                  n0w1_ref[...])
          + _dot_hi(onehot, ne0_ref[...]))
    o1 = (_dot(silu(_dot(h, n1w0_ref[...]) + n1b0_ref[...]),
                  n1w1_ref[...])
          + _dot_hi(onehot, ne1_ref[...]))
    gi = jnp.mean(h, axis=0, keepdims=True)
    og = (_dot_hi(silu(_dot_hi(gi, gw0_ref[...]) + gb0_ref[...]),
                  gw1_ref[...])
          + gb1_ref[...])

    o0_ref[...] = o0
    o1_ref[...] = o1
    og_ref[...] = og


@functools.partial(jax.jit, static_argnames=())
def kernel(nuclei, params, charges):
    p = params
    m0W0 = p['mp0']['W0']
    m1W0 = p['mp1']['W0']
    u0W0 = p['up0']['W0']
    u1W0 = p['up1']['W0']

    def row(b):
        return b.reshape(1, -1)

    f_row = row(p['rbf_f'])
    f_cat = jnp.tile(f_row, (1, _G))
    coef = jnp.sqrt(jnp.float32(2.0 / _CUTOFF)).reshape(1, 1)
    eye_g = jnp.eye(_G, dtype=jnp.float32)
    bd0 = jnp.kron(eye_g, m0W0[64:96])
    bd1 = jnp.kron(eye_g, m1W0[128:160])
    bdw0 = jnp.kron(eye_g, p['mp0']['W1'])
    bdw1 = jnp.kron(eye_g, p['mp1']['W1'])
    b1c0 = jnp.tile(row(p['mp0']['b1']), (1, _G))
    b1c1 = jnp.tile(row(p['mp1']['b1']), (1, _G))

    args = (
        nuclei.reshape(_N, 3),
        jnp.clip(charges.reshape(_N, 1).astype(jnp.int32), 0, 7),
        f_row, f_cat, coef,
        p['embed'],
        bd0, bd1, bdw0, bdw1, b1c0, b1c1,
        m0W0[0:32], m0W0[32:64], m0W0[64:96], row(p['mp0']['b0']),
        p['mp0']['W1'], row(p['mp0']['b1']),
        u0W0[0:32], u0W0[32:64], row(p['up0']['b0']),
        p['up0']['W1'], row(p['up0']['b1']),
        m1W0[0:64], m1W0[64:128], m1W0[128:160], row(p['mp1']['b0']),
        p['mp1']['W1'], row(p['mp1']['b1']),
        u1W0[0:64], u1W0[64:96], row(p['up1']['b0']),
        p['up1']['W1'], row(p['up1']['b1']),
        p['node0_mlp']['W0'], row(p['node0_mlp']['b0']),
        p['node0_mlp']['W1'], p['node0_embed'],
        p['node1_mlp']['W0'], row(p['node1_mlp']['b0']),
        p['node1_mlp']['W1'], p['node1_embed'],
        p['glob0_mlp']['W0'], row(p['glob0_mlp']['b0']),
        p['glob0_mlp']['W1'], row(p['glob0_mlp']['b1']),
    )

    o0, o1, og = pl.pallas_call(
        _tc_body,
        out_shape=(
            jax.ShapeDtypeStruct((_N, 1), jnp.float32),
            jax.ShapeDtypeStruct((_N, 3), jnp.float32),
            jax.ShapeDtypeStruct((1, 1), jnp.float32),
        ),
    )(*args)
    return (o0, o1, og.reshape(1))
